# baseline (speedup 1.0000x reference)
import functools

import jax
import jax.numpy as jnp
import numpy as np
from jax.experimental import pallas as pl
from jax.experimental.pallas import tpu as pltpu

SEQ_LEN = 336
PRED_LEN = 96
C_BLOCKS = 4
ZOO = 3
K = 2
BATCH = 32
CH = 128
KERNEL_W = 25
OUT_LEN = PRED_LEN * C_BLOCKS

PAIRS = [(0, 0), (0, 1), (0, 2), (1, 1), (1, 2), (2, 2)]
SPB = 32


def _movavg_matrix() -> np.ndarray:
    pad = (KERNEL_W - 1) // 2
    a = np.zeros((SEQ_LEN, SEQ_LEN), dtype=np.float64)
    for t in range(SEQ_LEN):
        for u in range(t - pad, t + pad + 1):
            a[t, min(max(u, 0), SEQ_LEN - 1)] += 1.0 / KERNEL_W
    return a.astype(np.float32)


_A = _movavg_matrix()


def _body(idx_ref, a_ref, w_ref, b_ref, data_ref, out_ref, mc_ref, dc_ref,
          rs_ref):
    pid = pl.program_id(0)

    @pl.when(pid == 0)
    def _compose():
        a = a_ref[...]
        eye = (jax.lax.broadcasted_iota(jnp.int32, (SEQ_LEN, SEQ_LEN), 0)
               == jax.lax.broadcasted_iota(jnp.int32, (SEQ_LEN, SEQ_LEN), 1)
               ).astype(jnp.float32)
        ms, ds = [], []
        for i in range(ZOO):
            cur_m = eye
            cur_b = jnp.zeros((SEQ_LEN, 1), dtype=jnp.float32)
            m_rows, d_rows = [], []
            for blk in range(C_BLOCKS):
                wse = w_ref[i, blk, 0]
                wtr = w_ref[i, blk, 1]
                bsum = b_ref[i, blk, 0] + b_ref[i, blk, 1]
                d = wse + jnp.dot(wtr - wse, a,
                                  preferred_element_type=jnp.float32)
                m_blk = jnp.dot(d, cur_m, preferred_element_type=jnp.float32)
                b_blk = jnp.dot(d, cur_b,
                                preferred_element_type=jnp.float32) + bsum
                m_rows.append(m_blk)
                d_rows.append(b_blk)
                cur_m = jnp.concatenate([cur_m[PRED_LEN:], m_blk], axis=0)
                cur_b = jnp.concatenate([cur_b[PRED_LEN:], b_blk], axis=0)
            ms.append(jnp.concatenate(m_rows, axis=0))
            ds.append(jnp.concatenate(d_rows, axis=0))
        for j, (lo, hi) in enumerate(PAIRS):
            mc = 0.5 * (ms[lo] + ms[hi])
            mc_ref[j] = mc.astype(jnp.bfloat16)
            dc_ref[j] = 0.5 * (ds[lo] + ds[hi])
            rs_ref[j] = jnp.sum(mc, axis=1, keepdims=True)

    @pl.when(pid > 0)
    def _apply():
        for s in range(SPB):
            b = (pid - 1) * SPB + s
            x = data_ref[s]
            s1 = jnp.sum(x, axis=0, keepdims=True)
            s2 = jnp.sum(x * x, axis=0, keepdims=True)
            mean = s1 * (1.0 / SEQ_LEN)
            var = s2 * (1.0 / SEQ_LEN) - mean * mean
            stdev = jnp.sqrt(var + 1e-5)
            e0 = idx_ref[0, b]
            e1 = idx_ref[1, b]
            lo = jnp.minimum(e0, e1)
            hi = jnp.maximum(e0, e1)
            sel = lo * ZOO - (lo * (lo + 1)) // 2 + hi
            mb = mc_ref[sel]
            y = jnp.dot(mb, x.astype(jnp.bfloat16),
                        preferred_element_type=jnp.float32)
            out_ref[s] = y + dc_ref[sel] * stdev + (1.0 - rs_ref[sel]) * mean


@functools.partial(jax.jit, static_argnames=("interpret",))
def _run(data, indices, ws, bs, interpret=False):
    a = jnp.asarray(_A)
    bs_col = bs.reshape(ZOO, C_BLOCKS, 2, PRED_LEN, 1)
    npair = len(PAIRS)
    out = pl.pallas_call(
        _body,
        grid=(1 + BATCH // SPB,),
        in_specs=[
            pl.BlockSpec(memory_space=pltpu.SMEM),
            pl.BlockSpec((SEQ_LEN, SEQ_LEN), lambda i: (0, 0)),
            pl.BlockSpec((ZOO, C_BLOCKS, 2, PRED_LEN, SEQ_LEN),
                         lambda i: (0, 0, 0, 0, 0)),
            pl.BlockSpec((ZOO, C_BLOCKS, 2, PRED_LEN, 1),
                         lambda i: (0, 0, 0, 0, 0)),
            pl.BlockSpec((SPB, SEQ_LEN, CH),
                         lambda i: (jnp.maximum(i - 1, 0), 0, 0)),
        ],
        out_specs=pl.BlockSpec((SPB, OUT_LEN, CH),
                               lambda i: (jnp.maximum(i - 1, 0), 0, 0)),
        out_shape=jax.ShapeDtypeStruct((BATCH, OUT_LEN, CH), jnp.float32),
        scratch_shapes=[
            pltpu.VMEM((npair, OUT_LEN, SEQ_LEN), jnp.bfloat16),
            pltpu.VMEM((npair, OUT_LEN, 1), jnp.float32),
            pltpu.VMEM((npair, OUT_LEN, 1), jnp.float32),
        ],
        interpret=interpret,
    )(indices, a, ws, bs_col, data)
    return out


def kernel(data, indices, x_mark_enc, x_dec, x_mark_dec, Ws, bs):
    return _run(data, indices.astype(jnp.int32), Ws, bs)


# speedup vs baseline: 1.2384x; 1.2384x over previous
import functools

import jax
import jax.numpy as jnp
from jax.experimental import pallas as pl
from jax.experimental.pallas import tpu as pltpu

SEQ_LEN = 336
PRED_LEN = 96
C_BLOCKS = 4
ZOO = 3
K = 2
BATCH = 32
CH = 128
KERNEL_W = 25
OUT_LEN = PRED_LEN * C_BLOCKS

PAIRS = [(0, 0), (0, 1), (0, 2), (1, 1), (1, 2), (2, 2)]
SPB = 16
AUG = SEQ_LEN + 2


def _movavg_matrix():
    pad = (KERNEL_W - 1) // 2
    ti = jax.lax.broadcasted_iota(jnp.int32, (SEQ_LEN, SEQ_LEN), 0)
    ci = jax.lax.broadcasted_iota(jnp.int32, (SEQ_LEN, SEQ_LEN), 1)
    band = (jnp.abs(ti - ci) <= pad).astype(jnp.float32)
    left = jnp.maximum(pad + 1 - ti, 0).astype(jnp.float32)
    right = jnp.maximum(ti - (SEQ_LEN - 2 - pad), 0).astype(jnp.float32)
    counts = jnp.where(ci == 0, left,
                       jnp.where(ci == SEQ_LEN - 1, right, band))
    return counts * (1.0 / KERNEL_W)


def _body(idx_ref, w_ref, b_ref, data_ref, out_ref, mc_ref):
    pid = pl.program_id(0)

    @pl.when(pid == 0)
    def _compose():
        a = _movavg_matrix()
        wall = w_ref[...]
        wse = wall[:, :, 0].reshape(ZOO * C_BLOCKS * PRED_LEN, SEQ_LEN)
        wdiff = (wall[:, :, 1] - wall[:, :, 0]).reshape(
            ZOO * C_BLOCKS * PRED_LEN, SEQ_LEN)
        dall = wse + jnp.dot(wdiff, a, preferred_element_type=jnp.float32)
        eye = (jax.lax.broadcasted_iota(jnp.int32, (SEQ_LEN, SEQ_LEN), 0)
               == jax.lax.broadcasted_iota(jnp.int32, (SEQ_LEN, SEQ_LEN), 1)
               ).astype(jnp.float32)
        ms, ds = [], []
        for i in range(ZOO):
            cur_m = eye
            cur_b = jnp.zeros((SEQ_LEN, 1), dtype=jnp.float32)
            m_rows, d_rows = [], []
            for blk in range(C_BLOCKS):
                r = (i * C_BLOCKS + blk) * PRED_LEN
                d = dall[r:r + PRED_LEN]
                bsum = b_ref[i, blk, 0] + b_ref[i, blk, 1]
                m_blk = jnp.dot(d, cur_m, preferred_element_type=jnp.float32)
                b_blk = jnp.dot(d, cur_b,
                                preferred_element_type=jnp.float32) + bsum
                m_rows.append(m_blk)
                d_rows.append(b_blk)
                cur_m = jnp.concatenate([cur_m[PRED_LEN:], m_blk], axis=0)
                cur_b = jnp.concatenate([cur_b[PRED_LEN:], b_blk], axis=0)
            ms.append(jnp.concatenate(m_rows, axis=0))
            ds.append(jnp.concatenate(d_rows, axis=0))
        for j, (lo, hi) in enumerate(PAIRS):
            mc = 0.5 * (ms[lo] + ms[hi])
            dc = 0.5 * (ds[lo] + ds[hi])
            rs1m = 1.0 - jnp.sum(mc, axis=1, keepdims=True)
            mc_ref[j] = jnp.concatenate([mc, dc, rs1m],
                                        axis=1).astype(jnp.bfloat16)

    @pl.when(pid > 0)
    def _apply():
        for s in range(SPB):
            b = (pid - 1) * SPB + s
            x = data_ref[s]
            s1 = jnp.sum(x, axis=0, keepdims=True)
            s2 = jnp.sum(x * x, axis=0, keepdims=True)
            mean = s1 * (1.0 / SEQ_LEN)
            var = s2 * (1.0 / SEQ_LEN) - mean * mean
            stdev = jnp.sqrt(var + 1e-5)
            e0 = idx_ref[0, b]
            e1 = idx_ref[1, b]
            lo = jnp.minimum(e0, e1)
            hi = jnp.maximum(e0, e1)
            sel = lo * ZOO - (lo * (lo + 1)) // 2 + hi
            xaug = jnp.concatenate([x, stdev, mean],
                                   axis=0).astype(jnp.bfloat16)
            out_ref[s] = jnp.dot(mc_ref[sel], xaug,
                                 preferred_element_type=jnp.float32)


@functools.partial(jax.jit, static_argnames=("interpret",))
def _run(data, indices, ws, bs, interpret=False):
    bs_col = bs.reshape(ZOO, C_BLOCKS, 2, PRED_LEN, 1)
    npair = len(PAIRS)
    out = pl.pallas_call(
        _body,
        grid=(1 + BATCH // SPB,),
        in_specs=[
            pl.BlockSpec(memory_space=pltpu.SMEM),
            pl.BlockSpec((ZOO, C_BLOCKS, 2, PRED_LEN, SEQ_LEN),
                         lambda i: (0, 0, 0, 0, 0)),
            pl.BlockSpec((ZOO, C_BLOCKS, 2, PRED_LEN, 1),
                         lambda i: (0, 0, 0, 0, 0)),
            pl.BlockSpec((SPB, SEQ_LEN, CH),
                         lambda i: (jnp.maximum(i - 1, 0), 0, 0)),
        ],
        out_specs=pl.BlockSpec((SPB, OUT_LEN, CH),
                               lambda i: (jnp.maximum(i - 1, 0), 0, 0)),
        out_shape=jax.ShapeDtypeStruct((BATCH, OUT_LEN, CH), jnp.float32),
        scratch_shapes=[
            pltpu.VMEM((npair, OUT_LEN, AUG), jnp.bfloat16),
        ],
        interpret=interpret,
    )(indices, ws, bs_col, data)
    return out


def kernel(data, indices, x_mark_enc, x_dec, x_mark_dec, Ws, bs):
    return _run(data, indices.astype(jnp.int32), Ws, bs)
